# baseline (speedup 1.0000x reference)
import jax
import jax.numpy as jnp
from jax.experimental import pallas as pl
from jax.experimental.pallas import tpu as pltpu

_B = 16384
_IN = 256
_HID = 256
_CD = 64
_NL = 4
_K = 1024
_BETA = 0.25
_R = 1024
_NBLK = _B // _R


def _rqvae_body(x_ref, w1_ref, b1_ref, w2_ref, b2_ref, cb_ref, cbt_ref,
                wd1_ref, bd1_ref, wd2_ref, bd2_ref,
                recon_ref, zq_ref, codes_ref,
                cb_out, commit_out, recon_out, total_out,
                qacc, racc):
    i = pl.program_id(0)

    @pl.when(i == 0)
    def _init():
        qacc[...] = jnp.zeros_like(qacc)
        racc[...] = jnp.zeros_like(racc)

    xb = x_ref[...]
    h = jnp.maximum(jnp.dot(xb, w1_ref[...]) + b1_ref[...], 0.0)
    z_e = jnp.dot(h, w2_ref[...]) + b2_ref[...]

    residual = z_e
    z_q = jnp.zeros_like(z_e)
    qsum = jnp.zeros((1, 1), jnp.float32)
    lane_iota = jax.lax.broadcasted_iota(jnp.int32, (_R, _K), 1)
    codes = []
    for level in range(_NL):
        cbt = cbt_ref[level]
        cb = cb_ref[level]
        rn = jnp.sum(residual ** 2, axis=1, keepdims=True)
        cn = jnp.sum(cbt ** 2, axis=0, keepdims=True)
        dist = rn + cn - 2.0 * jnp.dot(residual, cbt)
        min_d = jnp.min(dist, axis=1, keepdims=True)
        idx = jnp.min(jnp.where(dist == min_d, lane_iota, _K),
                      axis=1, keepdims=True)
        onehot = (lane_iota == idx).astype(jnp.float32)
        quantized = jnp.dot(onehot, cb,
                            precision=jax.lax.Precision.HIGHEST)
        t = quantized - residual
        qsum = qsum + jnp.sum(t * t, keepdims=True)
        codes.append(idx)
        q_st = residual + t
        z_q = z_q + q_st
        residual = residual - q_st

    hd = jnp.maximum(jnp.dot(z_q, wd1_ref[...]) + bd1_ref[...], 0.0)
    recon = jnp.dot(hd, wd2_ref[...]) + bd2_ref[...]

    recon_ref[...] = recon
    zq_ref[...] = z_q
    codes_ref[...] = jnp.concatenate(codes, axis=1)

    rdiff = recon - xb
    qacc[...] += qsum
    racc[...] += jnp.sum(rdiff * rdiff, keepdims=True)

    @pl.when(i == _NBLK - 1)
    def _final():
        cb_total = qacc[...] * (1.0 / (_B * _CD))
        rec = racc[...] * (1.0 / (_B * _IN))
        commit = _BETA * cb_total
        cb_out[...] = cb_total
        commit_out[...] = commit
        recon_out[...] = rec
        total_out[...] = cb_total + commit + rec


def kernel(x, W1, b1, W2, b2, codebooks, Wd1, bd1, Wd2, bd2):
    cbt = jnp.transpose(codebooks, (0, 2, 1))
    b1r = b1.reshape(1, _HID)
    b2r = b2.reshape(1, _CD)
    bd1r = bd1.reshape(1, _HID)
    bd2r = bd2.reshape(1, _IN)

    const = lambda *shape: pl.BlockSpec(shape, lambda i: tuple(0 for _ in shape))
    outs = pl.pallas_call(
        _rqvae_body,
        grid=(_NBLK,),
        in_specs=[
            pl.BlockSpec((_R, _IN), lambda i: (i, 0)),
            const(_IN, _HID), const(1, _HID),
            const(_HID, _CD), const(1, _CD),
            const(_NL, _K, _CD), const(_NL, _CD, _K),
            const(_CD, _HID), const(1, _HID),
            const(_HID, _IN), const(1, _IN),
        ],
        out_specs=[
            pl.BlockSpec((_R, _IN), lambda i: (i, 0)),
            pl.BlockSpec((_R, _CD), lambda i: (i, 0)),
            pl.BlockSpec((_R, _NL), lambda i: (i, 0)),
            const(1, 1), const(1, 1), const(1, 1), const(1, 1),
        ],
        out_shape=[
            jax.ShapeDtypeStruct((_B, _IN), jnp.float32),
            jax.ShapeDtypeStruct((_B, _CD), jnp.float32),
            jax.ShapeDtypeStruct((_B, _NL), jnp.int32),
            jax.ShapeDtypeStruct((1, 1), jnp.float32),
            jax.ShapeDtypeStruct((1, 1), jnp.float32),
            jax.ShapeDtypeStruct((1, 1), jnp.float32),
            jax.ShapeDtypeStruct((1, 1), jnp.float32),
        ],
        scratch_shapes=[
            pltpu.VMEM((1, 1), jnp.float32),
            pltpu.VMEM((1, 1), jnp.float32),
        ],
    )(x, W1, b1r, W2, b2r, codebooks, cbt, Wd1, bd1r, Wd2, bd2r)
    recon, z_q, codes, cb_t, commit_s, rec_l, total = outs
    return (recon, z_q, codes,
            cb_t.reshape(()), commit_s.reshape(()),
            rec_l.reshape(()), total.reshape(()))
